# Initial kernel scaffold; baseline (speedup 1.0000x reference)
import jax
import jax.numpy as jnp
from jax.experimental import pallas as pl


def kernel(spatial_encoder_weight, edge_dis_encoder_weight, spatial_types, shortest_path_types):
    raise NotImplementedError("write your pallas kernel here")


# speedup vs baseline: 169.0151x; 169.0151x over previous
import jax
import jax.numpy as jnp
from jax.experimental import pallas as pl
from jax.experimental.pallas import tpu as pltpu

_L = 4
_N = 256
_S = 16
_H = 8
_IB = 16
_JB = 128


def _body(spt_ref, st_ref, spw_ref, edw_ref, out_ref, spw_scr, edw_scr):
    first = ((pl.program_id(0) == 0) & (pl.program_id(1) == 0)
             & (pl.program_id(2) == 0))

    @pl.when(first)
    def _prep():
        zeros = jnp.zeros((_H, 128), jnp.float32)
        spw_scr[...] = zeros
        edw_scr[...] = zeros
        spw_scr[:, pl.ds(0, 68)] = spw_ref[...]
        edw_scr[:, pl.ds(0, 32)] = edw_ref[...]

    cnt = jnp.zeros((_IB, _JB), jnp.int32)
    accs = [jnp.zeros((_IB, _JB), jnp.float32) for _ in range(_H)]
    for s in range(_S):
        x = spt_ref[0, :, s, :]
        m = x >= 0
        cnt = cnt + m.astype(jnp.int32)
        safe = jnp.where(m, x, 32)
        for h in range(_H):
            tb = jnp.broadcast_to(edw_scr[h, :][None, :], (_IB, _JB))
            accs[h] = accs[h] + jnp.take_along_axis(
                tb, safe, axis=1, mode="promise_in_bounds")
    recip = 1.0 / (cnt.astype(jnp.float32) + 1e-6)
    stv = st_ref[0]
    for h in range(_H):
        tb = jnp.broadcast_to(spw_scr[h, :][None, :], (_IB, _JB))
        sp = jnp.take_along_axis(tb, stv, axis=1, mode="promise_in_bounds")
        out_ref[0, h] = sp + accs[h] * recip


def kernel(spatial_encoder_weight, edge_dis_encoder_weight, spatial_types,
           shortest_path_types):
    spt_t = jnp.transpose(shortest_path_types, (0, 1, 3, 2))
    spw_t = spatial_encoder_weight.T
    edw_t = edge_dis_encoder_weight.T

    out = pl.pallas_call(
        _body,
        grid=(_L, _N // _IB, _N // _JB),
        in_specs=[
            pl.BlockSpec((1, _IB, _S, _JB), lambda l, i, j: (l, i, 0, j)),
            pl.BlockSpec((1, _IB, _JB), lambda l, i, j: (l, i, j)),
            pl.BlockSpec((_H, 68), lambda l, i, j: (0, 0)),
            pl.BlockSpec((_H, 32), lambda l, i, j: (0, 0)),
        ],
        out_specs=pl.BlockSpec((1, _H, _IB, _JB),
                               lambda l, i, j: (l, 0, i, j)),
        out_shape=jax.ShapeDtypeStruct((_L, _H, _N, _N), jnp.float32),
        scratch_shapes=[pltpu.VMEM((_H, 128), jnp.float32),
                        pltpu.VMEM((_H, 128), jnp.float32)],
    )(spt_t, spatial_types, spw_t, edw_t)
    return out
